# baseline (speedup 1.0000x reference)
import jax
import jax.numpy as jnp
from jax.experimental import pallas as pl
from jax.experimental.pallas import tpu as pltpu

_KCHUNKS = 8
_LANES = 128


def _loss_kernel(x_ref, mu_ref, out_ref, acc_ref):
    j = pl.program_id(0)
    x = x_ref[...]
    mu = mu_ref[...]
    kc = mu.shape[1]
    dots = jnp.dot(x, mu, preferred_element_type=jnp.float32)
    mu_nsq = jnp.sum(mu * mu, axis=0, keepdims=True)
    score = mu_nsq - 2.0 * dots
    m = score[:, 0:_LANES]
    for t in range(1, kc // _LANES):
        m = jnp.minimum(m, score[:, t * _LANES:(t + 1) * _LANES])

    @pl.when(j == 0)
    def _():
        acc_ref[...] = m

    @pl.when(j > 0)
    def _():
        acc_ref[...] = jnp.minimum(acc_ref[...], m)

    @pl.when(j == _KCHUNKS - 1)
    def _():
        xsq = x * x
        ones = jnp.ones((x.shape[1], _LANES), jnp.float32)
        xn = jnp.dot(xsq, ones, preferred_element_type=jnp.float32)
        tot = acc_ref[...] + xn
        out_ref[...] = jnp.min(tot.T, axis=0) * (1.0 / x.shape[1])


def kernel(images, mu):
    B, G = images.shape
    K = mu.shape[1]
    kc = K // _KCHUNKS
    return pl.pallas_call(
        _loss_kernel,
        grid=(_KCHUNKS,),
        in_specs=[
            pl.BlockSpec((B, G), lambda j: (0, 0)),
            pl.BlockSpec((G, kc), lambda j: (0, j)),
        ],
        out_specs=pl.BlockSpec((B,), lambda j: (0,)),
        out_shape=jax.ShapeDtypeStruct((B,), jnp.float32),
        scratch_shapes=[pltpu.VMEM((B, _LANES), jnp.float32)],
    )(images, mu)


# speedup vs baseline: 2.0613x; 2.0613x over previous
import jax
import jax.numpy as jnp
from jax.experimental import pallas as pl
from jax.experimental.pallas import tpu as pltpu

_KCHUNKS = 2
_LANES = 128


def _loss_kernel(x_ref, mu_ref, out_ref, acc_ref):
    j = pl.program_id(0)
    x = x_ref[...]
    mu = mu_ref[...]
    kc = mu.shape[1]
    dots = jnp.dot(x, mu, preferred_element_type=jnp.float32)
    mu_nsq = jnp.sum(mu * mu, axis=0, keepdims=True)
    score = mu_nsq - 2.0 * dots
    m = score[:, 0:_LANES]
    for t in range(1, kc // _LANES):
        m = jnp.minimum(m, score[:, t * _LANES:(t + 1) * _LANES])

    @pl.when(j == 0)
    def _():
        acc_ref[...] = m

    @pl.when(j > 0)
    def _():
        acc_ref[...] = jnp.minimum(acc_ref[...], m)

    @pl.when(j == _KCHUNKS - 1)
    def _():
        xsq = x * x
        ones = jnp.ones((x.shape[1], _LANES), jnp.float32)
        xn = jnp.dot(xsq, ones, preferred_element_type=jnp.float32)
        tot = acc_ref[...] + xn
        out_ref[...] = jnp.min(tot.T, axis=0) * (1.0 / x.shape[1])


def kernel(images, mu):
    B, G = images.shape
    K = mu.shape[1]
    kc = K // _KCHUNKS
    return pl.pallas_call(
        _loss_kernel,
        grid=(_KCHUNKS,),
        in_specs=[
            pl.BlockSpec((B, G), lambda j: (0, 0)),
            pl.BlockSpec((G, kc), lambda j: (0, j)),
        ],
        out_specs=pl.BlockSpec((B,), lambda j: (0,)),
        out_shape=jax.ShapeDtypeStruct((B,), jnp.float32),
        scratch_shapes=[pltpu.VMEM((B, _LANES), jnp.float32)],
    )(images, mu)


# speedup vs baseline: 2.4031x; 1.1658x over previous
import jax
import jax.numpy as jnp
from jax.experimental import pallas as pl
from jax.experimental.pallas import tpu as pltpu

_KCHUNKS = 1
_LANES = 128


def _loss_kernel(x_ref, mu_ref, out_ref, acc_ref):
    j = pl.program_id(0)
    x = x_ref[...]
    mu = mu_ref[...]
    kc = mu.shape[1]
    dots = jnp.dot(x, mu, preferred_element_type=jnp.float32)
    mu_nsq = jnp.sum(mu * mu, axis=0, keepdims=True)
    score = mu_nsq - 2.0 * dots
    m = score[:, 0:_LANES]
    for t in range(1, kc // _LANES):
        m = jnp.minimum(m, score[:, t * _LANES:(t + 1) * _LANES])

    @pl.when(j == 0)
    def _():
        acc_ref[...] = m

    @pl.when(j > 0)
    def _():
        acc_ref[...] = jnp.minimum(acc_ref[...], m)

    @pl.when(j == _KCHUNKS - 1)
    def _():
        xsq = x * x
        ones = jnp.ones((x.shape[1], _LANES), jnp.float32)
        xn = jnp.dot(xsq, ones, preferred_element_type=jnp.float32)
        tot = acc_ref[...] + xn
        out_ref[...] = jnp.min(tot.T, axis=0) * (1.0 / x.shape[1])


def kernel(images, mu):
    B, G = images.shape
    K = mu.shape[1]
    kc = K // _KCHUNKS
    return pl.pallas_call(
        _loss_kernel,
        grid=(_KCHUNKS,),
        in_specs=[
            pl.BlockSpec((B, G), lambda j: (0, 0)),
            pl.BlockSpec((G, kc), lambda j: (0, j)),
        ],
        out_specs=pl.BlockSpec((B,), lambda j: (0,)),
        out_shape=jax.ShapeDtypeStruct((B,), jnp.float32),
        scratch_shapes=[pltpu.VMEM((B, _LANES), jnp.float32)],
    )(images, mu)
